# baseline (speedup 1.0000x reference)
import functools

import jax
import jax.numpy as jnp
from jax import lax
from jax.experimental import pallas as pl
from jax.experimental.pallas import tpu as pltpu
from jax.experimental.pallas import tpu_sc as plsc

_EPS = 1e-8
_BE = 512
_NC = 2
_NS = 16
_NW = _NC * _NS
_CG = 400
_CS = 200


def _prep_body(x_ref, wa_ref, wc_ref, b1_ref, ta_ref, tc_ref):
    xb = x_ref[...]
    ta_ref[...] = jnp.dot(xb, wa_ref[...], preferred_element_type=jnp.float32) + b1_ref[...]
    tc_ref[...] = jnp.dot(xb, wc_ref[...], preferred_element_type=jnp.float32)


def _prep_call(x, wa, wc, b1):
    n, d = x.shape
    bn = 1000
    return pl.pallas_call(
        _prep_body,
        grid=(n // bn,),
        in_specs=[
            pl.BlockSpec((bn, d), lambda i: (i, 0)),
            pl.BlockSpec((d, 128), lambda i: (0, 0)),
            pl.BlockSpec((d, 128), lambda i: (0, 0)),
            pl.BlockSpec((1, 128), lambda i: (0, 0)),
        ],
        out_specs=[
            pl.BlockSpec((bn, 128), lambda i: (i, 0)),
            pl.BlockSpec((bn, 128), lambda i: (i, 0)),
        ],
        out_shape=[
            jax.ShapeDtypeStruct((n, 128), jnp.float32),
            jax.ShapeDtypeStruct((n, 128), jnp.float32),
        ],
    )(x, wa, wc, b1)


def _gather_call(ta, tc, src, dst):
    e = src.shape[0]
    per_w = e // _NW
    nchunks = per_w // _CG
    f32 = jnp.float32
    mesh = plsc.VectorSubcoreMesh(core_axis_name="c", subcore_axis_name="s",
                                  num_cores=_NC, num_subcores=_NS)

    @functools.partial(
        pl.kernel, mesh=mesh,
        out_type=[
            jax.ShapeDtypeStruct((e, 128), f32),
            jax.ShapeDtypeStruct((e, 128), f32),
        ],
        scratch_types=[
            pltpu.VMEM((_CG,), jnp.int32),
            pltpu.VMEM((_CG,), jnp.int32),
            pltpu.VMEM((_CG, 128), f32),
            pltpu.VMEM((_CG, 128), f32),
            pltpu.SemaphoreType.DMA,
        ],
    )
    def gather_k(ta_hbm, tc_hbm, src_hbm, dst_hbm, ga_hbm, gc_hbm,
                 sidx, didx, abuf, cbuf, sem):
        wid = lax.axis_index("s") * _NC + lax.axis_index("c")
        base = wid * per_w

        def body(i, carry):
            off = pl.multiple_of(base + i * _CG, 8)
            pltpu.sync_copy(src_hbm.at[pl.ds(off, _CG)], sidx)
            pltpu.sync_copy(dst_hbm.at[pl.ds(off, _CG)], didx)
            c1 = pltpu.async_copy(ta_hbm.at[sidx], abuf, sem)
            c2 = pltpu.async_copy(tc_hbm.at[didx], cbuf, sem)
            c1.wait(); c2.wait()
            pltpu.sync_copy(abuf, ga_hbm.at[pl.ds(off, _CG)])
            pltpu.sync_copy(cbuf, gc_hbm.at[pl.ds(off, _CG)])
            return carry

        lax.fori_loop(0, nchunks, body, 0)

    return gather_k(ta, tc, src, dst)


def _aux_call(nrm8, src, dst, zc, ones_in, np_):
    e = src.shape[0]
    per_w = e // _NW
    nchunks = per_w // _CG
    per_tile_n = np_ // _NS
    f32 = jnp.float32
    mesh = plsc.VectorSubcoreMesh(core_axis_name="c", subcore_axis_name="s",
                                  num_cores=_NC, num_subcores=_NS)

    @functools.partial(
        pl.kernel, mesh=mesh,
        out_type=[
            jax.ShapeDtypeStruct((e, 8), f32),
            jax.ShapeDtypeStruct((e, 8), f32),
            jax.ShapeDtypeStruct((_NC, np_, 8), f32),
        ],
        scratch_types=[
            pltpu.VMEM((_CG,), jnp.int32),
            pltpu.VMEM((_CG,), jnp.int32),
            pltpu.VMEM((_CG, 8), f32),
            pltpu.VMEM((_CG, 8), f32),
            pltpu.VMEM((_CG, 8), f32),
            pltpu.VMEM_SHARED((np_, 8), f32),
            pltpu.SemaphoreType.DMA,
        ],
        compiler_params=pltpu.CompilerParams(use_tc_tiling_on_sc=False),
    )
    def aux_k(nrm_hbm, src_hbm, dst_hbm, zc_hbm, ones_hbm,
              ns_hbm, nd_hbm, cnt_hbm,
              sidx, didx, nsbuf, ndbuf, obuf, acc1, sem):
        cid = lax.axis_index("c")
        sid = lax.axis_index("s")
        wid = sid * _NC + cid
        base = wid * per_w
        row0 = sid * per_tile_n
        pltpu.sync_copy(ones_hbm, obuf)
        pltpu.sync_copy(zc_hbm, acc1.at[pl.ds(row0, per_tile_n)])
        plsc.subcore_barrier()

        def body(i, carry):
            off = pl.multiple_of(base + i * _CG, 8)
            pltpu.sync_copy(src_hbm.at[pl.ds(off, _CG)], sidx)
            pltpu.sync_copy(dst_hbm.at[pl.ds(off, _CG)], didx)
            c1 = pltpu.async_copy(nrm_hbm.at[sidx], nsbuf, sem)
            c2 = pltpu.async_copy(nrm_hbm.at[didx], ndbuf, sem)
            c1.wait(); c2.wait()
            pltpu.sync_copy(nsbuf, ns_hbm.at[pl.ds(off, _CG)])
            pltpu.sync_copy(ndbuf, nd_hbm.at[pl.ds(off, _CG)])
            pltpu.sync_copy(obuf, acc1.at[didx], add=True)
            return carry

        lax.fori_loop(0, nchunks, body, 0)
        plsc.subcore_barrier()
        pltpu.sync_copy(acc1.at[pl.ds(row0, per_tile_n)],
                        cnt_hbm.at[cid, pl.ds(row0, per_tile_n)])

    return aux_k(nrm8, src, dst, zc, ones_in)


def _scatter_call(msg, dst, z128, np_):
    e = msg.shape[0]
    per_w = e // _NW
    nchunks = per_w // _CS
    per_tile_n = np_ // _NS
    f32 = jnp.float32
    mesh = plsc.VectorSubcoreMesh(core_axis_name="c", subcore_axis_name="s",
                                  num_cores=_NC, num_subcores=_NS)

    @functools.partial(
        pl.kernel, mesh=mesh,
        out_type=jax.ShapeDtypeStruct((_NC, np_, 128), f32),
        scratch_types=[
            pltpu.VMEM((_CS,), jnp.int32),
            pltpu.VMEM((_CS, 128), f32),
            pltpu.VMEM_SHARED((np_, 128), f32),
            pltpu.SemaphoreType.DMA,
        ],
    )
    def scatter_k(msg_hbm, dst_hbm, z_hbm, out_hbm, didx, mbuf, acc, sem):
        cid = lax.axis_index("c")
        sid = lax.axis_index("s")
        wid = sid * _NC + cid
        base = wid * per_w
        row0 = sid * per_tile_n
        pltpu.sync_copy(z_hbm, acc.at[pl.ds(row0, per_tile_n)])
        plsc.subcore_barrier()

        def body(i, carry):
            off = pl.multiple_of(base + i * _CS, 8)
            pltpu.sync_copy(dst_hbm.at[pl.ds(off, _CS)], didx)
            pltpu.async_copy(msg_hbm.at[pl.ds(off, _CS)], mbuf, sem).wait()
            pltpu.sync_copy(mbuf, acc.at[didx], add=True)
            return carry

        lax.fori_loop(0, nchunks, body, 0)
        plsc.subcore_barrier()
        pltpu.sync_copy(acc.at[pl.ds(row0, per_tile_n)],
                        out_hbm.at[cid, pl.ds(row0, per_tile_n)])

    return scatter_k(msg, dst, z128)


def _combine_body(p_ref, c_ref, out_ref):
    p = p_ref[...]
    c = c_ref[...]
    cnt = c[0, :, 0:1] + c[1, :, 0:1]
    out_ref[...] = (p[0] + p[1]) / jnp.maximum(cnt, 1.0)


def _combine_call(partials, counts, n):
    bn = 1000
    return pl.pallas_call(
        _combine_body,
        grid=(n // bn,),
        in_specs=[
            pl.BlockSpec((2, bn, 128), lambda i: (0, i, 0)),
            pl.BlockSpec((2, bn, 8), lambda i: (0, i, 0)),
        ],
        out_specs=pl.BlockSpec((bn, 128), lambda i: (i, 0)),
        out_shape=jax.ShapeDtypeStruct((n, 128), jnp.float32),
    )(partials, counts)


def _mix(m, w_ref):
    blocks = []
    for h in range(3):
        acc = (w_ref[0, h] * m[0:3] + w_ref[1, h] * m[3:6]
               + w_ref[2, h] * m[6:9])
        blocks.append(acc)
    return jnp.concatenate(blocks, axis=0)


def _norms(m):
    sq = m * m
    rows = [sq[3 * c:3 * c + 1] + sq[3 * c + 1:3 * c + 2] + sq[3 * c + 2:3 * c + 3]
            for c in range(3)]
    return jnp.sqrt(jnp.maximum(jnp.concatenate(rows, axis=0), _EPS))


def _gate(v):
    n = _norms(v)
    sig = 1.0 / (1.0 + jnp.exp(-n))
    return jnp.concatenate(
        [v[3 * c:3 * c + 3] * sig[c:c + 1] for c in range(3)], axis=0)


def _vn_dot(vnt, w_ref):
    return lax.dot_general(vnt, w_ref[...], (((0,), (0,)), ((), ())),
                           preferred_element_type=jnp.float32)


def _dense_body(ga_ref, gc_ref, es_ref, ns_ref, nd_ref, ev_ref, eye_ref,
                wes_ref, wvn1_ref, w2s_ref, b2_ref, wvn2_ref,
                w3s_ref, b3_ref, wvn3_ref,
                wh1_ref, wv1_ref, wh2_ref, wv2_ref, wh3_ref,
                out_ref):
    f32 = jnp.float32
    s_pre = (ga_ref[...] + gc_ref[...]
             + jnp.dot(es_ref[...], wes_ref[...], preferred_element_type=f32))

    m_em = jnp.concatenate(
        [ns_ref[...][:, 0:3], ev_ref[...], nd_ref[...][:, 0:3]], axis=1)
    mv0 = lax.dot_general(m_em, eye_ref[...], (((0,), (0,)), ((), ())),
                          preferred_element_type=f32)
    vh1 = _mix(mv0, wh1_ref)
    vn1 = _norms(vh1)
    s1 = jnp.maximum(s_pre + _vn_dot(vn1, wvn1_ref), 0.0)
    v1 = _gate(_mix(vh1, wv1_ref))

    vh2 = _mix(v1, wh2_ref)
    vn2 = _norms(vh2)
    s2 = jnp.maximum(
        jnp.dot(s1, w2s_ref[...], preferred_element_type=f32)
        + _vn_dot(vn2, wvn2_ref) + b2_ref[...], 0.0)
    v2 = _gate(_mix(vh2, wv2_ref))

    vh3 = _mix(v2, wh3_ref)
    vn3 = _norms(vh3)
    out_ref[...] = (jnp.dot(s2, w3s_ref[...], preferred_element_type=f32)
                    + _vn_dot(vn3, wvn3_ref) + b3_ref[...])


def _dense_call(ga, gc, es, ns8, nd8, ev3, wes, wvn1, w2s, b2, wvn2,
                w3s, b3, wvn3, wh1, wv1, wh2, wv2, wh3):
    e = ga.shape[0]
    be = _BE
    eye = jnp.eye(be, dtype=jnp.float32)
    smem = lambda s: pl.BlockSpec(s, lambda i: (0, 0), memory_space=pltpu.SMEM)
    full = lambda s: pl.BlockSpec(s, lambda i: (0, 0))
    return pl.pallas_call(
        _dense_body,
        grid=(e // be,),
        in_specs=[
            pl.BlockSpec((be, 128), lambda i: (i, 0)),
            pl.BlockSpec((be, 128), lambda i: (i, 0)),
            pl.BlockSpec((be, 16), lambda i: (i, 0)),
            pl.BlockSpec((be, 8), lambda i: (i, 0)),
            pl.BlockSpec((be, 8), lambda i: (i, 0)),
            pl.BlockSpec((be, 3), lambda i: (i, 0)),
            full((be, be)),
            full((16, 128)), full((3, 128)),
            full((128, 128)), full((1, 128)), full((3, 128)),
            full((128, 128)), full((1, 128)), full((3, 128)),
            smem((3, 3)), smem((3, 3)), smem((3, 3)), smem((3, 3)),
            smem((3, 3)),
        ],
        out_specs=pl.BlockSpec((be, 128), lambda i: (i, 0)),
        out_shape=jax.ShapeDtypeStruct((e, 128), jnp.float32),
    )(ga, gc, es, ns8, nd8, ev3, eye, wes, wvn1, w2s, b2, wvn2, w3s, b3,
      wvn3, wh1, wv1, wh2, wv2, wh3)


def kernel(x, normals, edge_s, edge_v, edge_index,
           wh1, ws1_w, ws1_b, wv1, wh2, ws2_w, ws2_b, wv2,
           wh3, ws3_w, ws3_b, wv3):
    n, din = x.shape
    e = edge_s.shape[0]
    wa = ws1_w[0:128]
    wes = ws1_w[128:144]
    wc = ws1_w[144:272]
    wvn1 = ws1_w[272:275]
    w2s, wvn2 = ws2_w[0:128], ws2_w[128:131]
    w3s, wvn3 = ws3_w[0:128], ws3_w[128:131]
    b1 = ws1_b.reshape(1, 128)
    b2 = ws2_b.reshape(1, 128)
    b3 = ws3_b.reshape(1, 128)

    ta, tc = _prep_call(x, wa, wc, b1)

    src = edge_index[0]
    dst = edge_index[1]
    np_ = ((n + 8 * _NS - 1) // (8 * _NS)) * (8 * _NS)

    nrm8 = jnp.pad(normals, ((0, 0), (0, 5)))
    zc = jnp.zeros((np_ // _NS, 8), jnp.float32)
    ones_in = jnp.ones((_CG, 8), jnp.float32)

    ga, gc = _gather_call(ta, tc, src, dst)
    ns8, nd8, counts = _aux_call(nrm8, src, dst, zc, ones_in, np_)

    ev3 = edge_v.reshape(e, 3)

    msg = _dense_call(ga, gc, edge_s, ns8, nd8, ev3, wes, wvn1, w2s, b2,
                      wvn2, w3s, b3, wvn3, wh1, wv1, wh2, wv2, wh3)

    z128 = jnp.zeros((np_ // _NS, 128), jnp.float32)
    partials = _scatter_call(msg, dst, z128, np_)
    return _combine_call(partials, counts, n)


# speedup vs baseline: 1.0229x; 1.0229x over previous
import functools

import jax
import jax.numpy as jnp
from jax import lax
from jax.experimental import pallas as pl
from jax.experimental.pallas import tpu as pltpu
from jax.experimental.pallas import tpu_sc as plsc

_EPS = 1e-8
_BE = 640
_NC = 2
_NS = 16
_NW = _NC * _NS
_CG = 200
_CS = 200


def _prep_body(x_ref, wa_ref, wc_ref, b1_ref, ta_ref, tc_ref):
    xb = x_ref[...]
    ta_ref[...] = jnp.dot(xb, wa_ref[...], preferred_element_type=jnp.float32) + b1_ref[...]
    tc_ref[...] = jnp.dot(xb, wc_ref[...], preferred_element_type=jnp.float32)


def _prep_call(x, wa, wc, b1):
    n, d = x.shape
    bn = 1000
    return pl.pallas_call(
        _prep_body,
        grid=(n // bn,),
        in_specs=[
            pl.BlockSpec((bn, d), lambda i: (i, 0)),
            pl.BlockSpec((d, 128), lambda i: (0, 0)),
            pl.BlockSpec((d, 128), lambda i: (0, 0)),
            pl.BlockSpec((1, 128), lambda i: (0, 0)),
        ],
        out_specs=[
            pl.BlockSpec((bn, 128), lambda i: (i, 0)),
            pl.BlockSpec((bn, 128), lambda i: (i, 0)),
        ],
        out_shape=[
            jax.ShapeDtypeStruct((n, 128), jnp.float32),
            jax.ShapeDtypeStruct((n, 128), jnp.float32),
        ],
    )(x, wa, wc, b1)


def _gather_call(ta, tc, src, dst):
    e = src.shape[0]
    per_w = e // _NW
    nchunks = per_w // _CG
    f32 = jnp.float32
    mesh = plsc.VectorSubcoreMesh(core_axis_name="c", subcore_axis_name="s",
                                  num_cores=_NC, num_subcores=_NS)

    @functools.partial(
        pl.kernel, mesh=mesh,
        out_type=[
            jax.ShapeDtypeStruct((e, 128), f32),
            jax.ShapeDtypeStruct((e, 128), f32),
        ],
        scratch_types=[
            pltpu.VMEM((_CG,), jnp.int32),
            pltpu.VMEM((_CG,), jnp.int32),
            pltpu.VMEM((_CG, 128), f32),
            pltpu.VMEM((_CG, 128), f32),
            pltpu.SemaphoreType.DMA,
        ],
    )
    def gather_k(ta_hbm, tc_hbm, src_hbm, dst_hbm, ga_hbm, gc_hbm,
                 sidx, didx, abuf, cbuf, sem):
        wid = lax.axis_index("s") * _NC + lax.axis_index("c")
        base = wid * per_w

        def body(i, carry):
            off = pl.multiple_of(base + i * _CG, 8)
            pltpu.sync_copy(src_hbm.at[pl.ds(off, _CG)], sidx)
            pltpu.sync_copy(dst_hbm.at[pl.ds(off, _CG)], didx)
            c1 = pltpu.async_copy(ta_hbm.at[sidx], abuf, sem)
            c2 = pltpu.async_copy(tc_hbm.at[didx], cbuf, sem)
            c1.wait(); c2.wait()
            pltpu.sync_copy(abuf, ga_hbm.at[pl.ds(off, _CG)])
            pltpu.sync_copy(cbuf, gc_hbm.at[pl.ds(off, _CG)])
            return carry

        lax.fori_loop(0, nchunks, body, 0)

    return gather_k(ta, tc, src, dst)


def _aux_call(nrm8, src, dst, zc, ones_in, np_):
    e = src.shape[0]
    per_w = e // _NW
    nchunks = per_w // _CG
    per_tile_n = np_ // _NS
    f32 = jnp.float32
    mesh = plsc.VectorSubcoreMesh(core_axis_name="c", subcore_axis_name="s",
                                  num_cores=_NC, num_subcores=_NS)

    @functools.partial(
        pl.kernel, mesh=mesh,
        out_type=[
            jax.ShapeDtypeStruct((e, 8), f32),
            jax.ShapeDtypeStruct((e, 8), f32),
            jax.ShapeDtypeStruct((_NC, np_, 8), f32),
        ],
        scratch_types=[
            pltpu.VMEM((_CG,), jnp.int32),
            pltpu.VMEM((_CG,), jnp.int32),
            pltpu.VMEM((_CG, 8), f32),
            pltpu.VMEM((_CG, 8), f32),
            pltpu.VMEM((_CG, 8), f32),
            pltpu.VMEM_SHARED((np_, 8), f32),
            pltpu.SemaphoreType.DMA,
        ],
        compiler_params=pltpu.CompilerParams(use_tc_tiling_on_sc=False),
    )
    def aux_k(nrm_hbm, src_hbm, dst_hbm, zc_hbm, ones_hbm,
              ns_hbm, nd_hbm, cnt_hbm,
              sidx, didx, nsbuf, ndbuf, obuf, acc1, sem):
        cid = lax.axis_index("c")
        sid = lax.axis_index("s")
        wid = sid * _NC + cid
        base = wid * per_w
        row0 = sid * per_tile_n
        pltpu.sync_copy(ones_hbm, obuf)
        pltpu.sync_copy(zc_hbm, acc1.at[pl.ds(row0, per_tile_n)])
        plsc.subcore_barrier()

        def body(i, carry):
            off = pl.multiple_of(base + i * _CG, 8)
            pltpu.sync_copy(src_hbm.at[pl.ds(off, _CG)], sidx)
            pltpu.sync_copy(dst_hbm.at[pl.ds(off, _CG)], didx)
            c1 = pltpu.async_copy(nrm_hbm.at[sidx], nsbuf, sem)
            c2 = pltpu.async_copy(nrm_hbm.at[didx], ndbuf, sem)
            c1.wait(); c2.wait()
            pltpu.sync_copy(nsbuf, ns_hbm.at[pl.ds(off, _CG)])
            pltpu.sync_copy(ndbuf, nd_hbm.at[pl.ds(off, _CG)])
            pltpu.sync_copy(obuf, acc1.at[didx], add=True)
            return carry

        lax.fori_loop(0, nchunks, body, 0)
        plsc.subcore_barrier()
        pltpu.sync_copy(acc1.at[pl.ds(row0, per_tile_n)],
                        cnt_hbm.at[cid, pl.ds(row0, per_tile_n)])

    return aux_k(nrm8, src, dst, zc, ones_in)


def _scatter_call(msg, dst, z128, np_):
    e = msg.shape[0]
    per_w = e // _NW
    nchunks = per_w // _CS
    per_tile_n = np_ // _NS
    f32 = jnp.float32
    mesh = plsc.VectorSubcoreMesh(core_axis_name="c", subcore_axis_name="s",
                                  num_cores=_NC, num_subcores=_NS)

    @functools.partial(
        pl.kernel, mesh=mesh,
        out_type=jax.ShapeDtypeStruct((_NC, np_, 128), f32),
        scratch_types=[
            pltpu.VMEM((_CS,), jnp.int32),
            pltpu.VMEM((_CS, 128), f32),
            pltpu.VMEM_SHARED((np_, 128), f32),
            pltpu.SemaphoreType.DMA,
        ],
    )
    def scatter_k(msg_hbm, dst_hbm, z_hbm, out_hbm, didx, mbuf, acc, sem):
        cid = lax.axis_index("c")
        sid = lax.axis_index("s")
        wid = sid * _NC + cid
        base = wid * per_w
        row0 = sid * per_tile_n
        pltpu.sync_copy(z_hbm, acc.at[pl.ds(row0, per_tile_n)])
        plsc.subcore_barrier()

        def body(i, carry):
            off = pl.multiple_of(base + i * _CS, 8)
            pltpu.sync_copy(dst_hbm.at[pl.ds(off, _CS)], didx)
            pltpu.async_copy(msg_hbm.at[pl.ds(off, _CS)], mbuf, sem).wait()
            pltpu.sync_copy(mbuf, acc.at[didx], add=True)
            return carry

        lax.fori_loop(0, nchunks, body, 0)
        plsc.subcore_barrier()
        pltpu.sync_copy(acc.at[pl.ds(row0, per_tile_n)],
                        out_hbm.at[cid, pl.ds(row0, per_tile_n)])

    return scatter_k(msg, dst, z128)


def _combine_body(p1_ref, p2_ref, c1_ref, c2_ref, out_ref):
    p1, p2 = p1_ref[...], p2_ref[...]
    c1, c2 = c1_ref[...], c2_ref[...]
    cnt = c1[0, :, 0:1] + c1[1, :, 0:1] + c2[0, :, 0:1] + c2[1, :, 0:1]
    out_ref[...] = (p1[0] + p1[1] + p2[0] + p2[1]) / jnp.maximum(cnt, 1.0)


def _combine_call(p1, p2, c1, c2, n):
    bn = 1000
    return pl.pallas_call(
        _combine_body,
        grid=(n // bn,),
        in_specs=[
            pl.BlockSpec((2, bn, 128), lambda i: (0, i, 0)),
            pl.BlockSpec((2, bn, 128), lambda i: (0, i, 0)),
            pl.BlockSpec((2, bn, 8), lambda i: (0, i, 0)),
            pl.BlockSpec((2, bn, 8), lambda i: (0, i, 0)),
        ],
        out_specs=pl.BlockSpec((bn, 128), lambda i: (i, 0)),
        out_shape=jax.ShapeDtypeStruct((n, 128), jnp.float32),
    )(p1, p2, c1, c2)


def _mix(m, w_ref):
    blocks = []
    for h in range(3):
        acc = (w_ref[0, h] * m[0:3] + w_ref[1, h] * m[3:6]
               + w_ref[2, h] * m[6:9])
        blocks.append(acc)
    return jnp.concatenate(blocks, axis=0)


def _norms(m):
    sq = m * m
    rows = [sq[3 * c:3 * c + 1] + sq[3 * c + 1:3 * c + 2] + sq[3 * c + 2:3 * c + 3]
            for c in range(3)]
    return jnp.sqrt(jnp.maximum(jnp.concatenate(rows, axis=0), _EPS))


def _gate(v):
    n = _norms(v)
    sig = 1.0 / (1.0 + jnp.exp(-n))
    return jnp.concatenate(
        [v[3 * c:3 * c + 3] * sig[c:c + 1] for c in range(3)], axis=0)


def _vn_dot(vnt, w_ref):
    return lax.dot_general(vnt, w_ref[...], (((0,), (0,)), ((), ())),
                           preferred_element_type=jnp.float32)


def _dense_body(ga_ref, gc_ref, es_ref, ns_ref, nd_ref, ev_ref,
                wes_ref, wvn1_ref, w2s_ref, b2_ref, wvn2_ref,
                w3s_ref, b3_ref, wvn3_ref,
                wh1_ref, wv1_ref, wh2_ref, wv2_ref, wh3_ref,
                out_ref):
    f32 = jnp.float32
    s_pre = (ga_ref[...] + gc_ref[...]
             + jnp.dot(es_ref[...], wes_ref[...], preferred_element_type=f32))

    m_em = jnp.concatenate(
        [ns_ref[...][:, 0:3], ev_ref[...], nd_ref[...][:, 0:3]], axis=1)
    mv0 = m_em.T
    vh1 = _mix(mv0, wh1_ref)
    vn1 = _norms(vh1)
    s1 = jnp.maximum(s_pre + _vn_dot(vn1, wvn1_ref), 0.0)
    v1 = _gate(_mix(vh1, wv1_ref))

    vh2 = _mix(v1, wh2_ref)
    vn2 = _norms(vh2)
    s2 = jnp.maximum(
        jnp.dot(s1, w2s_ref[...], preferred_element_type=f32)
        + _vn_dot(vn2, wvn2_ref) + b2_ref[...], 0.0)
    v2 = _gate(_mix(vh2, wv2_ref))

    vh3 = _mix(v2, wh3_ref)
    vn3 = _norms(vh3)
    out_ref[...] = (jnp.dot(s2, w3s_ref[...], preferred_element_type=f32)
                    + _vn_dot(vn3, wvn3_ref) + b3_ref[...])


def _dense_call(ga, gc, es, ns8, nd8, ev3, wes, wvn1, w2s, b2, wvn2,
                w3s, b3, wvn3, wh1, wv1, wh2, wv2, wh3):
    e = ga.shape[0]
    be = _BE
    smem = lambda s: pl.BlockSpec(s, lambda i: (0, 0), memory_space=pltpu.SMEM)
    full = lambda s: pl.BlockSpec(s, lambda i: (0, 0))
    return pl.pallas_call(
        _dense_body,
        grid=(e // be,),
        in_specs=[
            pl.BlockSpec((be, 128), lambda i: (i, 0)),
            pl.BlockSpec((be, 128), lambda i: (i, 0)),
            pl.BlockSpec((be, 16), lambda i: (i, 0)),
            pl.BlockSpec((be, 8), lambda i: (i, 0)),
            pl.BlockSpec((be, 8), lambda i: (i, 0)),
            pl.BlockSpec((be, 3), lambda i: (i, 0)),
            full((16, 128)), full((3, 128)),
            full((128, 128)), full((1, 128)), full((3, 128)),
            full((128, 128)), full((1, 128)), full((3, 128)),
            smem((3, 3)), smem((3, 3)), smem((3, 3)), smem((3, 3)),
            smem((3, 3)),
        ],
        out_specs=pl.BlockSpec((be, 128), lambda i: (i, 0)),
        out_shape=jax.ShapeDtypeStruct((e, 128), jnp.float32),
    )(ga, gc, es, ns8, nd8, ev3, wes, wvn1, w2s, b2, wvn2, w3s, b3,
      wvn3, wh1, wv1, wh2, wv2, wh3)


def kernel(x, normals, edge_s, edge_v, edge_index,
           wh1, ws1_w, ws1_b, wv1, wh2, ws2_w, ws2_b, wv2,
           wh3, ws3_w, ws3_b, wv3):
    n, din = x.shape
    e = edge_s.shape[0]
    wa = ws1_w[0:128]
    wes = ws1_w[128:144]
    wc = ws1_w[144:272]
    wvn1 = ws1_w[272:275]
    w2s, wvn2 = ws2_w[0:128], ws2_w[128:131]
    w3s, wvn3 = ws3_w[0:128], ws3_w[128:131]
    b1 = ws1_b.reshape(1, 128)
    b2 = ws2_b.reshape(1, 128)
    b3 = ws3_b.reshape(1, 128)

    ta, tc = _prep_call(x, wa, wc, b1)

    src = edge_index[0]
    dst = edge_index[1]
    np_ = ((n + 8 * _NS - 1) // (8 * _NS)) * (8 * _NS)

    nrm8 = jnp.pad(normals, ((0, 0), (0, 5)))
    zc = jnp.zeros((np_ // _NS, 8), jnp.float32)
    ones_in = jnp.ones((_CG, 8), jnp.float32)
    z128 = jnp.zeros((np_ // _NS, 128), jnp.float32)
    ev3 = edge_v.reshape(e, 3)

    h = e // 2
    parts, cnts = [], []
    msgs = []
    for lo in (0, h):
        s_h, d_h = src[lo:lo + h], dst[lo:lo + h]
        ga, gc = _gather_call(ta, tc, s_h, d_h)
        ns8, nd8, counts = _aux_call(nrm8, s_h, d_h, zc, ones_in, np_)
        msg = _dense_call(ga, gc, edge_s[lo:lo + h], ns8, nd8,
                          ev3[lo:lo + h], wes, wvn1, w2s, b2, wvn2, w3s, b3,
                          wvn3, wh1, wv1, wh2, wv2, wh3)
        parts.append(_scatter_call(msg, d_h, z128, np_))
        cnts.append(counts)
    return _combine_call(parts[0], parts[1], cnts[0], cnts[1], n)


# speedup vs baseline: 1.1259x; 1.1006x over previous
import functools

import jax
import jax.numpy as jnp
from jax import lax
from jax.experimental import pallas as pl
from jax.experimental.pallas import tpu as pltpu
from jax.experimental.pallas import tpu_sc as plsc

_EPS = 1e-8
_BE = 640
_NC = 2
_NS = 16
_NW = _NC * _NS
_CG = 200
_CS = 200


def _prep_body(x_ref, wa_ref, wc_ref, b1_ref, ta_ref, tc_ref):
    xb = x_ref[...]
    ta_ref[...] = jnp.dot(xb, wa_ref[...], preferred_element_type=jnp.float32) + b1_ref[...]
    tc_ref[...] = jnp.dot(xb, wc_ref[...], preferred_element_type=jnp.float32)


def _prep_call(x, wa, wc, b1):
    n, d = x.shape
    bn = 1000
    return pl.pallas_call(
        _prep_body,
        grid=(n // bn,),
        in_specs=[
            pl.BlockSpec((bn, d), lambda i: (i, 0)),
            pl.BlockSpec((d, 128), lambda i: (0, 0)),
            pl.BlockSpec((d, 128), lambda i: (0, 0)),
            pl.BlockSpec((1, 128), lambda i: (0, 0)),
        ],
        out_specs=[
            pl.BlockSpec((bn, 128), lambda i: (i, 0)),
            pl.BlockSpec((bn, 128), lambda i: (i, 0)),
        ],
        out_shape=[
            jax.ShapeDtypeStruct((n, 128), jnp.float32),
            jax.ShapeDtypeStruct((n, 128), jnp.float32),
        ],
    )(x, wa, wc, b1)


def _gather_call(ta, tc, src, dst):
    e = src.shape[0]
    per_w = e // _NW
    nchunks = per_w // _CG
    f32 = jnp.float32
    mesh = plsc.VectorSubcoreMesh(core_axis_name="c", subcore_axis_name="s",
                                  num_cores=_NC, num_subcores=_NS)

    @functools.partial(
        pl.kernel, mesh=mesh,
        out_type=[
            jax.ShapeDtypeStruct((e, 128), f32),
            jax.ShapeDtypeStruct((e, 128), f32),
        ],
        scratch_types=[
            pltpu.VMEM((_CG,), jnp.int32),
            pltpu.VMEM((_CG,), jnp.int32),
            pltpu.VMEM((_CG, 128), f32),
            pltpu.VMEM((_CG, 128), f32),
            pltpu.SemaphoreType.DMA,
        ],
    )
    def gather_k(ta_hbm, tc_hbm, src_hbm, dst_hbm, ga_hbm, gc_hbm,
                 sidx, didx, abuf, cbuf, sem):
        wid = lax.axis_index("s") * _NC + lax.axis_index("c")
        base = wid * per_w

        def body(i, carry):
            off = pl.multiple_of(base + i * _CG, 8)
            pltpu.sync_copy(src_hbm.at[pl.ds(off, _CG)], sidx)
            pltpu.sync_copy(dst_hbm.at[pl.ds(off, _CG)], didx)
            c1 = pltpu.async_copy(ta_hbm.at[sidx], abuf, sem)
            c2 = pltpu.async_copy(tc_hbm.at[didx], cbuf, sem)
            c1.wait(); c2.wait()
            pltpu.sync_copy(abuf, ga_hbm.at[pl.ds(off, _CG)])
            pltpu.sync_copy(cbuf, gc_hbm.at[pl.ds(off, _CG)])
            return carry

        lax.fori_loop(0, nchunks, body, 0)

    return gather_k(ta, tc, src, dst)


def _aux_call(nrm8, src, dst, zc, ones_in, np_):
    e = src.shape[0]
    per_w = e // _NW
    nchunks = per_w // _CG
    per_tile_n = np_ // _NS
    f32 = jnp.float32
    mesh = plsc.VectorSubcoreMesh(core_axis_name="c", subcore_axis_name="s",
                                  num_cores=_NC, num_subcores=_NS)

    @functools.partial(
        pl.kernel, mesh=mesh,
        out_type=[
            jax.ShapeDtypeStruct((e, 8), f32),
            jax.ShapeDtypeStruct((e, 8), f32),
            jax.ShapeDtypeStruct((_NC, np_, 8), f32),
        ],
        scratch_types=[
            pltpu.VMEM((_CG,), jnp.int32),
            pltpu.VMEM((_CG,), jnp.int32),
            pltpu.VMEM((_CG, 8), f32),
            pltpu.VMEM((_CG, 8), f32),
            pltpu.VMEM((_CG, 8), f32),
            pltpu.VMEM_SHARED((np_, 8), f32),
            pltpu.SemaphoreType.DMA,
        ],
        compiler_params=pltpu.CompilerParams(use_tc_tiling_on_sc=False),
    )
    def aux_k(nrm_hbm, src_hbm, dst_hbm, zc_hbm, ones_hbm,
              ns_hbm, nd_hbm, cnt_hbm,
              sidx, didx, nsbuf, ndbuf, obuf, acc1, sem):
        cid = lax.axis_index("c")
        sid = lax.axis_index("s")
        wid = sid * _NC + cid
        base = wid * per_w
        row0 = sid * per_tile_n
        pltpu.sync_copy(ones_hbm, obuf)
        pltpu.sync_copy(zc_hbm, acc1.at[pl.ds(row0, per_tile_n)])
        plsc.subcore_barrier()

        def body(i, carry):
            off = pl.multiple_of(base + i * _CG, 8)
            pltpu.sync_copy(src_hbm.at[pl.ds(off, _CG)], sidx)
            pltpu.sync_copy(dst_hbm.at[pl.ds(off, _CG)], didx)
            c1 = pltpu.async_copy(nrm_hbm.at[sidx], nsbuf, sem)
            c2 = pltpu.async_copy(nrm_hbm.at[didx], ndbuf, sem)
            c1.wait(); c2.wait()
            pltpu.sync_copy(nsbuf, ns_hbm.at[pl.ds(off, _CG)])
            pltpu.sync_copy(ndbuf, nd_hbm.at[pl.ds(off, _CG)])
            pltpu.sync_copy(obuf, acc1.at[didx], add=True)
            return carry

        lax.fori_loop(0, nchunks, body, 0)
        plsc.subcore_barrier()
        pltpu.sync_copy(acc1.at[pl.ds(row0, per_tile_n)],
                        cnt_hbm.at[cid, pl.ds(row0, per_tile_n)])

    return aux_k(nrm8, src, dst, zc, ones_in)


def _scatter_call(msg, dst, z128, np_):
    e = msg.shape[0]
    per_w = e // _NW
    nchunks = per_w // _CS
    per_tile_n = np_ // _NS
    f32 = jnp.float32
    mesh = plsc.VectorSubcoreMesh(core_axis_name="c", subcore_axis_name="s",
                                  num_cores=_NC, num_subcores=_NS)

    @functools.partial(
        pl.kernel, mesh=mesh,
        out_type=jax.ShapeDtypeStruct((_NC, np_, 128), f32),
        scratch_types=[
            pltpu.VMEM((_CS,), jnp.int32),
            pltpu.VMEM((_CS, 128), f32),
            pltpu.VMEM_SHARED((np_, 128), f32),
            pltpu.SemaphoreType.DMA,
        ],
    )
    def scatter_k(msg_hbm, dst_hbm, z_hbm, out_hbm, didx, mbuf, acc, sem):
        cid = lax.axis_index("c")
        sid = lax.axis_index("s")
        wid = sid * _NC + cid
        base = wid * per_w
        row0 = sid * per_tile_n
        pltpu.sync_copy(z_hbm, acc.at[pl.ds(row0, per_tile_n)])
        plsc.subcore_barrier()

        def body(i, carry):
            off = pl.multiple_of(base + i * _CS, 8)
            pltpu.sync_copy(dst_hbm.at[pl.ds(off, _CS)], didx)
            pltpu.async_copy(msg_hbm.at[pl.ds(off, _CS)], mbuf, sem).wait()
            pltpu.sync_copy(mbuf, acc.at[didx], add=True)
            return carry

        lax.fori_loop(0, nchunks, body, 0)
        plsc.subcore_barrier()
        pltpu.sync_copy(acc.at[pl.ds(row0, per_tile_n)],
                        out_hbm.at[cid, pl.ds(row0, per_tile_n)])

    return scatter_k(msg, dst, z128)


def _combine_body(p1_ref, p2_ref, c1_ref, c2_ref, out_ref):
    p1, p2 = p1_ref[...], p2_ref[...]
    c1, c2 = c1_ref[...], c2_ref[...]
    cnt = c1[0, :, 0:1] + c1[1, :, 0:1] + c2[0, :, 0:1] + c2[1, :, 0:1]
    out_ref[...] = (p1[0] + p1[1] + p2[0] + p2[1]) / jnp.maximum(cnt, 1.0)


def _combine_call(p1, p2, c1, c2, n):
    bn = 1000
    return pl.pallas_call(
        _combine_body,
        grid=(n // bn,),
        in_specs=[
            pl.BlockSpec((2, bn, 128), lambda i: (0, i, 0)),
            pl.BlockSpec((2, bn, 128), lambda i: (0, i, 0)),
            pl.BlockSpec((2, bn, 8), lambda i: (0, i, 0)),
            pl.BlockSpec((2, bn, 8), lambda i: (0, i, 0)),
        ],
        out_specs=pl.BlockSpec((bn, 128), lambda i: (i, 0)),
        out_shape=jax.ShapeDtypeStruct((n, 128), jnp.float32),
    )(p1, p2, c1, c2)


def _mix(m, w_ref):
    blocks = []
    for h in range(3):
        acc = (w_ref[0, h] * m[0:3] + w_ref[1, h] * m[3:6]
               + w_ref[2, h] * m[6:9])
        blocks.append(acc)
    return jnp.concatenate(blocks, axis=0)


def _norms(m):
    sq = m * m
    rows = [sq[3 * c:3 * c + 1] + sq[3 * c + 1:3 * c + 2] + sq[3 * c + 2:3 * c + 3]
            for c in range(3)]
    return jnp.sqrt(jnp.maximum(jnp.concatenate(rows, axis=0), _EPS))


def _gate(v):
    n = _norms(v)
    sig = 1.0 / (1.0 + jnp.exp(-n))
    return jnp.concatenate(
        [v[3 * c:3 * c + 3] * sig[c:c + 1] for c in range(3)], axis=0)


def _vn_dot(vnt, w_ref):
    return lax.dot_general(vnt, w_ref[...], (((0,), (0,)), ((), ())),
                           preferred_element_type=jnp.float32)


def _dense_body(ga_ref, gc_ref, es_ref, ns_ref, nd_ref, ev_ref,
                wes_ref, wvn1_ref, w2s_ref, b2_ref, wvn2_ref,
                w3s_ref, b3_ref, wvn3_ref,
                wh1_ref, wv1_ref, wh2_ref, wv2_ref, wh3_ref,
                out_ref):
    f32 = jnp.float32
    s_pre = (ga_ref[...] + gc_ref[...]
             + jnp.dot(es_ref[...], wes_ref[...], preferred_element_type=f32))

    m_em = jnp.concatenate(
        [ns_ref[...][:, 0:3], ev_ref[...], nd_ref[...][:, 0:3]], axis=1)
    mv0 = m_em.T
    vh1 = _mix(mv0, wh1_ref)
    vn1 = _norms(vh1)
    s1 = jnp.maximum(s_pre + _vn_dot(vn1, wvn1_ref), 0.0)
    v1 = _gate(_mix(vh1, wv1_ref))

    vh2 = _mix(v1, wh2_ref)
    vn2 = _norms(vh2)
    s2 = jnp.maximum(
        jnp.dot(s1, w2s_ref[...], preferred_element_type=f32)
        + _vn_dot(vn2, wvn2_ref) + b2_ref[...], 0.0)
    v2 = _gate(_mix(vh2, wv2_ref))

    vh3 = _mix(v2, wh3_ref)
    vn3 = _norms(vh3)
    out_ref[...] = (jnp.dot(s2, w3s_ref[...], preferred_element_type=f32)
                    + _vn_dot(vn3, wvn3_ref) + b3_ref[...])


def _dense_call(ga, gc, es, ns8, nd8, ev3, wes, wvn1, w2s, b2, wvn2,
                w3s, b3, wvn3, wh1, wv1, wh2, wv2, wh3, blk0):
    e = ga.shape[0]
    be = _BE
    smem = lambda s: pl.BlockSpec(s, lambda i: (0, 0), memory_space=pltpu.SMEM)
    full = lambda s: pl.BlockSpec(s, lambda i: (0, 0))
    return pl.pallas_call(
        _dense_body,
        grid=(e // be,),
        in_specs=[
            pl.BlockSpec((be, 128), lambda i: (i, 0)),
            pl.BlockSpec((be, 128), lambda i: (i, 0)),
            pl.BlockSpec((be, 16), lambda i, blk0=blk0: (i + blk0, 0)),
            pl.BlockSpec((be, 8), lambda i: (i, 0)),
            pl.BlockSpec((be, 8), lambda i: (i, 0)),
            pl.BlockSpec((be, 3), lambda i, blk0=blk0: (i + blk0, 0)),
            full((16, 128)), full((3, 128)),
            full((128, 128)), full((1, 128)), full((3, 128)),
            full((128, 128)), full((1, 128)), full((3, 128)),
            smem((3, 3)), smem((3, 3)), smem((3, 3)), smem((3, 3)),
            smem((3, 3)),
        ],
        out_specs=pl.BlockSpec((be, 128), lambda i: (i, 0)),
        out_shape=jax.ShapeDtypeStruct((e, 128), jnp.float32),
    )(ga, gc, es, ns8, nd8, ev3, wes, wvn1, w2s, b2, wvn2, w3s, b3,
      wvn3, wh1, wv1, wh2, wv2, wh3)


def kernel(x, normals, edge_s, edge_v, edge_index,
           wh1, ws1_w, ws1_b, wv1, wh2, ws2_w, ws2_b, wv2,
           wh3, ws3_w, ws3_b, wv3):
    n, din = x.shape
    e = edge_s.shape[0]
    wa = ws1_w[0:128]
    wes = ws1_w[128:144]
    wc = ws1_w[144:272]
    wvn1 = ws1_w[272:275]
    w2s, wvn2 = ws2_w[0:128], ws2_w[128:131]
    w3s, wvn3 = ws3_w[0:128], ws3_w[128:131]
    b1 = ws1_b.reshape(1, 128)
    b2 = ws2_b.reshape(1, 128)
    b3 = ws3_b.reshape(1, 128)

    ta, tc = _prep_call(x, wa, wc, b1)

    src = edge_index[0]
    dst = edge_index[1]
    np_ = ((n + 8 * _NS - 1) // (8 * _NS)) * (8 * _NS)

    nrm8 = jnp.pad(normals, ((0, 0), (0, 5)))
    zc = jnp.zeros((np_ // _NS, 8), jnp.float32)
    ones_in = jnp.ones((_CG, 8), jnp.float32)
    z128 = jnp.zeros((np_ // _NS, 128), jnp.float32)
    ev3 = edge_v.reshape(e, 3)

    h = e // 2
    parts, cnts = [], []
    msgs = []
    for lo in (0, h):
        s_h, d_h = src[lo:lo + h], dst[lo:lo + h]
        ga, gc = _gather_call(ta, tc, s_h, d_h)
        ns8, nd8, counts = _aux_call(nrm8, s_h, d_h, zc, ones_in, np_)
        msg = _dense_call(ga, gc, edge_s, ns8, nd8,
                          ev3, wes, wvn1, w2s, b2, wvn2, w3s, b3,
                          wvn3, wh1, wv1, wh2, wv2, wh3, lo // _BE)
        parts.append(_scatter_call(msg, d_h, z128, np_))
        cnts.append(counts)
    return _combine_call(parts[0], parts[1], cnts[0], cnts[1], n)


# speedup vs baseline: 1.1951x; 1.0615x over previous
import functools

import jax
import jax.numpy as jnp
from jax import lax
from jax.experimental import pallas as pl
from jax.experimental.pallas import tpu as pltpu
from jax.experimental.pallas import tpu_sc as plsc

_EPS = 1e-8
_BE = 1000
_NC = 2
_NS = 16
_NW = _NC * _NS
_CG = 200
_CS = 200


def _prep_body(x_ref, wa_ref, wc_ref, b1_ref, ta_ref, tc_ref):
    xb = x_ref[...]
    ta_ref[...] = jnp.dot(xb, wa_ref[...], preferred_element_type=jnp.float32) + b1_ref[...]
    tc_ref[...] = jnp.dot(xb, wc_ref[...], preferred_element_type=jnp.float32)


def _prep_call(x, wa, wc, b1):
    n, d = x.shape
    bn = 1000
    return pl.pallas_call(
        _prep_body,
        grid=(n // bn,),
        in_specs=[
            pl.BlockSpec((bn, d), lambda i: (i, 0)),
            pl.BlockSpec((d, 128), lambda i: (0, 0)),
            pl.BlockSpec((d, 128), lambda i: (0, 0)),
            pl.BlockSpec((1, 128), lambda i: (0, 0)),
        ],
        out_specs=[
            pl.BlockSpec((bn, 128), lambda i: (i, 0)),
            pl.BlockSpec((bn, 128), lambda i: (i, 0)),
        ],
        out_shape=[
            jax.ShapeDtypeStruct((n, 128), jnp.float32),
            jax.ShapeDtypeStruct((n, 128), jnp.float32),
        ],
    )(x, wa, wc, b1)


def _gather_call(ta, tc, src, dst):
    e = src.shape[0]
    per_w = e // _NW
    nchunks = per_w // _CG
    f32 = jnp.float32
    mesh = plsc.VectorSubcoreMesh(core_axis_name="c", subcore_axis_name="s",
                                  num_cores=_NC, num_subcores=_NS)

    @functools.partial(
        pl.kernel, mesh=mesh,
        out_type=[
            jax.ShapeDtypeStruct((e, 128), f32),
            jax.ShapeDtypeStruct((e, 128), f32),
        ],
        scratch_types=[
            pltpu.VMEM((_CG,), jnp.int32),
            pltpu.VMEM((_CG,), jnp.int32),
            pltpu.VMEM((_CG, 128), f32),
            pltpu.VMEM((_CG, 128), f32),
            pltpu.SemaphoreType.DMA,
        ],
    )
    def gather_k(ta_hbm, tc_hbm, src_hbm, dst_hbm, ga_hbm, gc_hbm,
                 sidx, didx, abuf, cbuf, sem):
        wid = lax.axis_index("s") * _NC + lax.axis_index("c")
        base = wid * per_w

        def body(i, carry):
            off = pl.multiple_of(base + i * _CG, 8)
            pltpu.sync_copy(src_hbm.at[pl.ds(off, _CG)], sidx)
            pltpu.sync_copy(dst_hbm.at[pl.ds(off, _CG)], didx)
            c1 = pltpu.async_copy(ta_hbm.at[sidx], abuf, sem)
            c2 = pltpu.async_copy(tc_hbm.at[didx], cbuf, sem)
            c1.wait(); c2.wait()
            pltpu.sync_copy(abuf, ga_hbm.at[pl.ds(off, _CG)])
            pltpu.sync_copy(cbuf, gc_hbm.at[pl.ds(off, _CG)])
            return carry

        lax.fori_loop(0, nchunks, body, 0)

    return gather_k(ta, tc, src, dst)


def _aux_call(nrm8, src, dst, zc, ones_in, np_):
    e = src.shape[0]
    per_w = e // _NW
    nchunks = per_w // _CG
    per_tile_n = np_ // _NS
    f32 = jnp.float32
    mesh = plsc.VectorSubcoreMesh(core_axis_name="c", subcore_axis_name="s",
                                  num_cores=_NC, num_subcores=_NS)

    @functools.partial(
        pl.kernel, mesh=mesh,
        out_type=[
            jax.ShapeDtypeStruct((e, 8), f32),
            jax.ShapeDtypeStruct((e, 8), f32),
            jax.ShapeDtypeStruct((_NC, np_, 8), f32),
        ],
        scratch_types=[
            pltpu.VMEM((_CG,), jnp.int32),
            pltpu.VMEM((_CG,), jnp.int32),
            pltpu.VMEM((_CG, 8), f32),
            pltpu.VMEM((_CG, 8), f32),
            pltpu.VMEM((_CG, 8), f32),
            pltpu.VMEM_SHARED((np_, 8), f32),
            pltpu.SemaphoreType.DMA,
        ],
        compiler_params=pltpu.CompilerParams(use_tc_tiling_on_sc=False),
    )
    def aux_k(nrm_hbm, src_hbm, dst_hbm, zc_hbm, ones_hbm,
              ns_hbm, nd_hbm, cnt_hbm,
              sidx, didx, nsbuf, ndbuf, obuf, acc1, sem):
        cid = lax.axis_index("c")
        sid = lax.axis_index("s")
        wid = sid * _NC + cid
        base = wid * per_w
        row0 = sid * per_tile_n
        pltpu.sync_copy(ones_hbm, obuf)
        pltpu.sync_copy(zc_hbm, acc1.at[pl.ds(row0, per_tile_n)])
        plsc.subcore_barrier()

        def body(i, carry):
            off = pl.multiple_of(base + i * _CG, 8)
            pltpu.sync_copy(src_hbm.at[pl.ds(off, _CG)], sidx)
            pltpu.sync_copy(dst_hbm.at[pl.ds(off, _CG)], didx)
            c1 = pltpu.async_copy(nrm_hbm.at[sidx], nsbuf, sem)
            c2 = pltpu.async_copy(nrm_hbm.at[didx], ndbuf, sem)
            c1.wait(); c2.wait()
            pltpu.sync_copy(nsbuf, ns_hbm.at[pl.ds(off, _CG)])
            pltpu.sync_copy(ndbuf, nd_hbm.at[pl.ds(off, _CG)])
            pltpu.sync_copy(obuf, acc1.at[didx], add=True)
            return carry

        lax.fori_loop(0, nchunks, body, 0)
        plsc.subcore_barrier()
        pltpu.sync_copy(acc1.at[pl.ds(row0, per_tile_n)],
                        cnt_hbm.at[cid, pl.ds(row0, per_tile_n)])

    return aux_k(nrm8, src, dst, zc, ones_in)


def _scatter_call(msg, dst, z128, np_):
    e = msg.shape[0]
    per_w = e // _NW
    nchunks = per_w // _CS
    per_tile_n = np_ // _NS
    f32 = jnp.float32
    mesh = plsc.VectorSubcoreMesh(core_axis_name="c", subcore_axis_name="s",
                                  num_cores=_NC, num_subcores=_NS)

    @functools.partial(
        pl.kernel, mesh=mesh,
        out_type=jax.ShapeDtypeStruct((_NC, np_, 128), f32),
        scratch_types=[
            pltpu.VMEM((_CS,), jnp.int32),
            pltpu.VMEM((_CS, 128), f32),
            pltpu.VMEM_SHARED((np_, 128), f32),
            pltpu.SemaphoreType.DMA,
        ],
    )
    def scatter_k(msg_hbm, dst_hbm, z_hbm, out_hbm, didx, mbuf, acc, sem):
        cid = lax.axis_index("c")
        sid = lax.axis_index("s")
        wid = sid * _NC + cid
        base = wid * per_w
        row0 = sid * per_tile_n
        pltpu.sync_copy(z_hbm, acc.at[pl.ds(row0, per_tile_n)])
        plsc.subcore_barrier()

        def body(i, carry):
            off = pl.multiple_of(base + i * _CS, 8)
            pltpu.sync_copy(dst_hbm.at[pl.ds(off, _CS)], didx)
            pltpu.async_copy(msg_hbm.at[pl.ds(off, _CS)], mbuf, sem).wait()
            pltpu.sync_copy(mbuf, acc.at[didx], add=True)
            return carry

        lax.fori_loop(0, nchunks, body, 0)
        plsc.subcore_barrier()
        pltpu.sync_copy(acc.at[pl.ds(row0, per_tile_n)],
                        out_hbm.at[cid, pl.ds(row0, per_tile_n)])

    return scatter_k(msg, dst, z128)


def _combine_body(p1_ref, p2_ref, c1_ref, c2_ref, out_ref):
    p1, p2 = p1_ref[...], p2_ref[...]
    c1, c2 = c1_ref[...], c2_ref[...]
    cnt = c1[0, :, 0:1] + c1[1, :, 0:1] + c2[0, :, 0:1] + c2[1, :, 0:1]
    out_ref[...] = (p1[0] + p1[1] + p2[0] + p2[1]) / jnp.maximum(cnt, 1.0)


def _combine_call(p1, p2, c1, c2, n):
    bn = 1000
    return pl.pallas_call(
        _combine_body,
        grid=(n // bn,),
        in_specs=[
            pl.BlockSpec((2, bn, 128), lambda i: (0, i, 0)),
            pl.BlockSpec((2, bn, 128), lambda i: (0, i, 0)),
            pl.BlockSpec((2, bn, 8), lambda i: (0, i, 0)),
            pl.BlockSpec((2, bn, 8), lambda i: (0, i, 0)),
        ],
        out_specs=pl.BlockSpec((bn, 128), lambda i: (i, 0)),
        out_shape=jax.ShapeDtypeStruct((n, 128), jnp.float32),
    )(p1, p2, c1, c2)


def _mix(m, w_ref):
    blocks = []
    for h in range(3):
        acc = (w_ref[0, h] * m[0:3] + w_ref[1, h] * m[3:6]
               + w_ref[2, h] * m[6:9])
        blocks.append(acc)
    return jnp.concatenate(blocks, axis=0)


def _norms(m):
    sq = m * m
    rows = [sq[3 * c:3 * c + 1] + sq[3 * c + 1:3 * c + 2] + sq[3 * c + 2:3 * c + 3]
            for c in range(3)]
    return jnp.sqrt(jnp.maximum(jnp.concatenate(rows, axis=0), _EPS))


def _gate(v):
    n = _norms(v)
    sig = 1.0 / (1.0 + jnp.exp(-n))
    return jnp.concatenate(
        [v[3 * c:3 * c + 3] * sig[c:c + 1] for c in range(3)], axis=0)


def _vn_dot(vnt, w_ref):
    return lax.dot_general(vnt, w_ref[...], (((0,), (0,)), ((), ())),
                           preferred_element_type=jnp.float32)


def _dense_body(ga_ref, gc_ref, es_ref, ns_ref, nd_ref, ev_ref,
                wes_ref, wvn1_ref, w2s_ref, b2_ref, wvn2_ref,
                w3s_ref, b3_ref, wvn3_ref,
                wh1_ref, wv1_ref, wh2_ref, wv2_ref, wh3_ref,
                out_ref):
    f32 = jnp.float32
    s_pre = (ga_ref[...] + gc_ref[...]
             + jnp.dot(es_ref[...], wes_ref[...], preferred_element_type=f32))

    m_em = jnp.concatenate(
        [ns_ref[...][:, 0:3], ev_ref[...], nd_ref[...][:, 0:3]], axis=1)
    mv0 = m_em.T
    vh1 = _mix(mv0, wh1_ref)
    vn1 = _norms(vh1)
    s1 = jnp.maximum(s_pre + _vn_dot(vn1, wvn1_ref), 0.0)
    v1 = _gate(_mix(vh1, wv1_ref))

    vh2 = _mix(v1, wh2_ref)
    vn2 = _norms(vh2)
    s2 = jnp.maximum(
        jnp.dot(s1, w2s_ref[...], preferred_element_type=f32)
        + _vn_dot(vn2, wvn2_ref) + b2_ref[...], 0.0)
    v2 = _gate(_mix(vh2, wv2_ref))

    vh3 = _mix(v2, wh3_ref)
    vn3 = _norms(vh3)
    out_ref[...] = (jnp.dot(s2, w3s_ref[...], preferred_element_type=f32)
                    + _vn_dot(vn3, wvn3_ref) + b3_ref[...])


def _dense_call(ga, gc, es, ns8, nd8, ev3, wes, wvn1, w2s, b2, wvn2,
                w3s, b3, wvn3, wh1, wv1, wh2, wv2, wh3, blk0):
    e = ga.shape[0]
    be = _BE
    smem = lambda s: pl.BlockSpec(s, lambda i: (0, 0), memory_space=pltpu.SMEM)
    full = lambda s: pl.BlockSpec(s, lambda i: (0, 0))
    return pl.pallas_call(
        _dense_body,
        grid=(e // be,),
        in_specs=[
            pl.BlockSpec((be, 128), lambda i: (i, 0)),
            pl.BlockSpec((be, 128), lambda i: (i, 0)),
            pl.BlockSpec((be, 16), lambda i, blk0=blk0: (i + blk0, 0)),
            pl.BlockSpec((be, 8), lambda i: (i, 0)),
            pl.BlockSpec((be, 8), lambda i: (i, 0)),
            pl.BlockSpec((be, 3), lambda i, blk0=blk0: (i + blk0, 0)),
            full((16, 128)), full((3, 128)),
            full((128, 128)), full((1, 128)), full((3, 128)),
            full((128, 128)), full((1, 128)), full((3, 128)),
            smem((3, 3)), smem((3, 3)), smem((3, 3)), smem((3, 3)),
            smem((3, 3)),
        ],
        out_specs=pl.BlockSpec((be, 128), lambda i: (i, 0)),
        out_shape=jax.ShapeDtypeStruct((e, 128), jnp.float32),
    )(ga, gc, es, ns8, nd8, ev3, wes, wvn1, w2s, b2, wvn2, w3s, b3,
      wvn3, wh1, wv1, wh2, wv2, wh3)


def kernel(x, normals, edge_s, edge_v, edge_index,
           wh1, ws1_w, ws1_b, wv1, wh2, ws2_w, ws2_b, wv2,
           wh3, ws3_w, ws3_b, wv3):
    n, din = x.shape
    e = edge_s.shape[0]
    wa = ws1_w[0:128]
    wes = ws1_w[128:144]
    wc = ws1_w[144:272]
    wvn1 = ws1_w[272:275]
    w2s, wvn2 = ws2_w[0:128], ws2_w[128:131]
    w3s, wvn3 = ws3_w[0:128], ws3_w[128:131]
    b1 = ws1_b.reshape(1, 128)
    b2 = ws2_b.reshape(1, 128)
    b3 = ws3_b.reshape(1, 128)

    ta, tc = _prep_call(x, wa, wc, b1)

    src = edge_index[0]
    dst = edge_index[1]
    np_ = ((n + 8 * _NS - 1) // (8 * _NS)) * (8 * _NS)

    nrm8 = jnp.pad(normals, ((0, 0), (0, 5)))
    zc = jnp.zeros((np_ // _NS, 8), jnp.float32)
    ones_in = jnp.ones((_CG, 8), jnp.float32)
    z128 = jnp.zeros((np_ // _NS, 128), jnp.float32)
    ev3 = edge_v.reshape(e, 3)

    h = e // 2
    parts, cnts = [], []
    msgs = []
    for lo in (0, h):
        s_h, d_h = src[lo:lo + h], dst[lo:lo + h]
        ga, gc = _gather_call(ta, tc, s_h, d_h)
        ns8, nd8, counts = _aux_call(nrm8, s_h, d_h, zc, ones_in, np_)
        msg = _dense_call(ga, gc, edge_s, ns8, nd8,
                          ev3, wes, wvn1, w2s, b2, wvn2, w3s, b3,
                          wvn3, wh1, wv1, wh2, wv2, wh3, lo // _BE)
        parts.append(_scatter_call(msg, d_h, z128, np_))
        cnts.append(counts)
    return _combine_call(parts[0], parts[1], cnts[0], cnts[1], n)


# speedup vs baseline: 1.3253x; 1.1089x over previous
import functools

import jax
import jax.numpy as jnp
from jax import lax
from jax.experimental import pallas as pl
from jax.experimental.pallas import tpu as pltpu
from jax.experimental.pallas import tpu_sc as plsc

_EPS = 1e-8
_BE = 2000
_NC = 2
_NS = 16
_NW = _NC * _NS
_CG = 200
_CS = 200


def _prep_body(x_ref, wa_ref, wc_ref, b1_ref, ta_ref, tc_ref):
    xb = x_ref[...]
    ta_ref[...] = jnp.dot(xb, wa_ref[...], preferred_element_type=jnp.float32) + b1_ref[...]
    tc_ref[...] = jnp.dot(xb, wc_ref[...], preferred_element_type=jnp.float32)


def _prep_call(x, wa, wc, b1):
    n, d = x.shape
    bn = 1000
    return pl.pallas_call(
        _prep_body,
        grid=(n // bn,),
        in_specs=[
            pl.BlockSpec((bn, d), lambda i: (i, 0)),
            pl.BlockSpec((d, 128), lambda i: (0, 0)),
            pl.BlockSpec((d, 128), lambda i: (0, 0)),
            pl.BlockSpec((1, 128), lambda i: (0, 0)),
        ],
        out_specs=[
            pl.BlockSpec((bn, 128), lambda i: (i, 0)),
            pl.BlockSpec((bn, 128), lambda i: (i, 0)),
        ],
        out_shape=[
            jax.ShapeDtypeStruct((n, 128), jnp.float32),
            jax.ShapeDtypeStruct((n, 128), jnp.float32),
        ],
    )(x, wa, wc, b1)


def _gather_call(ta, tc, src, dst):
    e = src.shape[0]
    per_w = e // _NW
    nchunks = per_w // _CG
    f32 = jnp.float32
    mesh = plsc.VectorSubcoreMesh(core_axis_name="c", subcore_axis_name="s",
                                  num_cores=_NC, num_subcores=_NS)

    @functools.partial(
        pl.kernel, mesh=mesh,
        out_type=[
            jax.ShapeDtypeStruct((e, 128), f32),
            jax.ShapeDtypeStruct((e, 128), f32),
        ],
        scratch_types=[
            pltpu.VMEM((_CG,), jnp.int32),
            pltpu.VMEM((_CG,), jnp.int32),
            pltpu.VMEM((_CG, 128), f32),
            pltpu.VMEM((_CG, 128), f32),
            pltpu.SemaphoreType.DMA,
        ],
    )
    def gather_k(ta_hbm, tc_hbm, src_hbm, dst_hbm, ga_hbm, gc_hbm,
                 sidx, didx, abuf, cbuf, sem):
        wid = lax.axis_index("s") * _NC + lax.axis_index("c")
        base = wid * per_w

        def body(i, carry):
            off = pl.multiple_of(base + i * _CG, 8)
            pltpu.sync_copy(src_hbm.at[pl.ds(off, _CG)], sidx)
            pltpu.sync_copy(dst_hbm.at[pl.ds(off, _CG)], didx)
            c1 = pltpu.async_copy(ta_hbm.at[sidx], abuf, sem)
            c2 = pltpu.async_copy(tc_hbm.at[didx], cbuf, sem)
            c1.wait(); c2.wait()
            pltpu.sync_copy(abuf, ga_hbm.at[pl.ds(off, _CG)])
            pltpu.sync_copy(cbuf, gc_hbm.at[pl.ds(off, _CG)])
            return carry

        lax.fori_loop(0, nchunks, body, 0)

    return gather_k(ta, tc, src, dst)


def _aux_call(nrm8, src, dst, zc, ones_in, np_):
    e = src.shape[0]
    per_w = e // _NW
    nchunks = per_w // _CG
    per_tile_n = np_ // _NS
    f32 = jnp.float32
    mesh = plsc.VectorSubcoreMesh(core_axis_name="c", subcore_axis_name="s",
                                  num_cores=_NC, num_subcores=_NS)

    @functools.partial(
        pl.kernel, mesh=mesh,
        out_type=[
            jax.ShapeDtypeStruct((e, 8), f32),
            jax.ShapeDtypeStruct((e, 8), f32),
            jax.ShapeDtypeStruct((_NC, np_, 8), f32),
        ],
        scratch_types=[
            pltpu.VMEM((_CG,), jnp.int32),
            pltpu.VMEM((_CG,), jnp.int32),
            pltpu.VMEM((_CG, 8), f32),
            pltpu.VMEM((_CG, 8), f32),
            pltpu.VMEM((_CG, 8), f32),
            pltpu.VMEM_SHARED((np_, 8), f32),
            pltpu.SemaphoreType.DMA,
        ],
        compiler_params=pltpu.CompilerParams(use_tc_tiling_on_sc=False),
    )
    def aux_k(nrm_hbm, src_hbm, dst_hbm, zc_hbm, ones_hbm,
              ns_hbm, nd_hbm, cnt_hbm,
              sidx, didx, nsbuf, ndbuf, obuf, acc1, sem):
        cid = lax.axis_index("c")
        sid = lax.axis_index("s")
        wid = sid * _NC + cid
        base = wid * per_w
        row0 = sid * per_tile_n
        pltpu.sync_copy(ones_hbm, obuf)
        pltpu.sync_copy(zc_hbm, acc1.at[pl.ds(row0, per_tile_n)])
        plsc.subcore_barrier()

        def body(i, carry):
            off = pl.multiple_of(base + i * _CG, 8)
            pltpu.sync_copy(src_hbm.at[pl.ds(off, _CG)], sidx)
            pltpu.sync_copy(dst_hbm.at[pl.ds(off, _CG)], didx)
            c1 = pltpu.async_copy(nrm_hbm.at[sidx], nsbuf, sem)
            c2 = pltpu.async_copy(nrm_hbm.at[didx], ndbuf, sem)
            c1.wait(); c2.wait()
            pltpu.sync_copy(nsbuf, ns_hbm.at[pl.ds(off, _CG)])
            pltpu.sync_copy(ndbuf, nd_hbm.at[pl.ds(off, _CG)])
            pltpu.sync_copy(obuf, acc1.at[didx], add=True)
            return carry

        lax.fori_loop(0, nchunks, body, 0)
        plsc.subcore_barrier()
        pltpu.sync_copy(acc1.at[pl.ds(row0, per_tile_n)],
                        cnt_hbm.at[cid, pl.ds(row0, per_tile_n)])

    return aux_k(nrm8, src, dst, zc, ones_in)


def _scatter_call(msg, dst, z128, np_):
    e = msg.shape[0]
    per_w = e // _NW
    nchunks = per_w // _CS
    per_tile_n = np_ // _NS
    f32 = jnp.float32
    mesh = plsc.VectorSubcoreMesh(core_axis_name="c", subcore_axis_name="s",
                                  num_cores=_NC, num_subcores=_NS)

    @functools.partial(
        pl.kernel, mesh=mesh,
        out_type=jax.ShapeDtypeStruct((_NC, np_, 128), f32),
        scratch_types=[
            pltpu.VMEM((_CS,), jnp.int32),
            pltpu.VMEM((_CS, 128), f32),
            pltpu.VMEM_SHARED((np_, 128), f32),
            pltpu.SemaphoreType.DMA,
        ],
    )
    def scatter_k(msg_hbm, dst_hbm, z_hbm, out_hbm, didx, mbuf, acc, sem):
        cid = lax.axis_index("c")
        sid = lax.axis_index("s")
        wid = sid * _NC + cid
        base = wid * per_w
        row0 = sid * per_tile_n
        pltpu.sync_copy(z_hbm, acc.at[pl.ds(row0, per_tile_n)])
        plsc.subcore_barrier()

        def body(i, carry):
            off = pl.multiple_of(base + i * _CS, 8)
            pltpu.sync_copy(dst_hbm.at[pl.ds(off, _CS)], didx)
            pltpu.async_copy(msg_hbm.at[pl.ds(off, _CS)], mbuf, sem).wait()
            pltpu.sync_copy(mbuf, acc.at[didx], add=True)
            return carry

        lax.fori_loop(0, nchunks, body, 0)
        plsc.subcore_barrier()
        pltpu.sync_copy(acc.at[pl.ds(row0, per_tile_n)],
                        out_hbm.at[cid, pl.ds(row0, per_tile_n)])

    return scatter_k(msg, dst, z128)


def _combine_body(p1_ref, p2_ref, c1_ref, c2_ref, out_ref):
    p1, p2 = p1_ref[...], p2_ref[...]
    c1, c2 = c1_ref[...], c2_ref[...]
    cnt = c1[0, :, 0:1] + c1[1, :, 0:1] + c2[0, :, 0:1] + c2[1, :, 0:1]
    out_ref[...] = (p1[0] + p1[1] + p2[0] + p2[1]) / jnp.maximum(cnt, 1.0)


def _combine_call(p1, p2, c1, c2, n):
    bn = 1000
    return pl.pallas_call(
        _combine_body,
        grid=(n // bn,),
        in_specs=[
            pl.BlockSpec((2, bn, 128), lambda i: (0, i, 0)),
            pl.BlockSpec((2, bn, 128), lambda i: (0, i, 0)),
            pl.BlockSpec((2, bn, 8), lambda i: (0, i, 0)),
            pl.BlockSpec((2, bn, 8), lambda i: (0, i, 0)),
        ],
        out_specs=pl.BlockSpec((bn, 128), lambda i: (i, 0)),
        out_shape=jax.ShapeDtypeStruct((n, 128), jnp.float32),
    )(p1, p2, c1, c2)


def _mix(m, w_ref):
    blocks = []
    for h in range(3):
        acc = (w_ref[0, h] * m[0:3] + w_ref[1, h] * m[3:6]
               + w_ref[2, h] * m[6:9])
        blocks.append(acc)
    return jnp.concatenate(blocks, axis=0)


def _norms(m):
    sq = m * m
    rows = [sq[3 * c:3 * c + 1] + sq[3 * c + 1:3 * c + 2] + sq[3 * c + 2:3 * c + 3]
            for c in range(3)]
    return jnp.sqrt(jnp.maximum(jnp.concatenate(rows, axis=0), _EPS))


def _gate(v):
    n = _norms(v)
    sig = 1.0 / (1.0 + jnp.exp(-n))
    return jnp.concatenate(
        [v[3 * c:3 * c + 3] * sig[c:c + 1] for c in range(3)], axis=0)


def _vn_dot(vnt, w_ref):
    return lax.dot_general(vnt, w_ref[...], (((0,), (0,)), ((), ())),
                           preferred_element_type=jnp.float32)


def _dense_body(ga_ref, gc_ref, es_ref, ns_ref, nd_ref, ev_ref,
                wes_ref, wvn1_ref, w2s_ref, b2_ref, wvn2_ref,
                w3s_ref, b3_ref, wvn3_ref,
                wh1_ref, wv1_ref, wh2_ref, wv2_ref, wh3_ref,
                out_ref):
    f32 = jnp.float32
    s_pre = (ga_ref[...] + gc_ref[...]
             + jnp.dot(es_ref[...], wes_ref[...], preferred_element_type=f32))

    m_em = jnp.concatenate(
        [ns_ref[...][:, 0:3], ev_ref[...], nd_ref[...][:, 0:3]], axis=1)
    mv0 = m_em.T
    vh1 = _mix(mv0, wh1_ref)
    vn1 = _norms(vh1)
    s1 = jnp.maximum(s_pre + _vn_dot(vn1, wvn1_ref), 0.0)
    v1 = _gate(_mix(vh1, wv1_ref))

    vh2 = _mix(v1, wh2_ref)
    vn2 = _norms(vh2)
    s2 = jnp.maximum(
        jnp.dot(s1, w2s_ref[...], preferred_element_type=f32)
        + _vn_dot(vn2, wvn2_ref) + b2_ref[...], 0.0)
    v2 = _gate(_mix(vh2, wv2_ref))

    vh3 = _mix(v2, wh3_ref)
    vn3 = _norms(vh3)
    out_ref[...] = (jnp.dot(s2, w3s_ref[...], preferred_element_type=f32)
                    + _vn_dot(vn3, wvn3_ref) + b3_ref[...])


def _dense_call(ga, gc, es, ns8, nd8, ev3, wes, wvn1, w2s, b2, wvn2,
                w3s, b3, wvn3, wh1, wv1, wh2, wv2, wh3, blk0):
    e = ga.shape[0]
    be = _BE
    smem = lambda s: pl.BlockSpec(s, lambda i: (0, 0), memory_space=pltpu.SMEM)
    full = lambda s: pl.BlockSpec(s, lambda i: (0, 0))
    return pl.pallas_call(
        _dense_body,
        grid=(e // be,),
        in_specs=[
            pl.BlockSpec((be, 128), lambda i: (i, 0)),
            pl.BlockSpec((be, 128), lambda i: (i, 0)),
            pl.BlockSpec((be, 16), lambda i, blk0=blk0: (i + blk0, 0)),
            pl.BlockSpec((be, 8), lambda i: (i, 0)),
            pl.BlockSpec((be, 8), lambda i: (i, 0)),
            pl.BlockSpec((be, 3), lambda i, blk0=blk0: (i + blk0, 0)),
            full((16, 128)), full((3, 128)),
            full((128, 128)), full((1, 128)), full((3, 128)),
            full((128, 128)), full((1, 128)), full((3, 128)),
            smem((3, 3)), smem((3, 3)), smem((3, 3)), smem((3, 3)),
            smem((3, 3)),
        ],
        out_specs=pl.BlockSpec((be, 128), lambda i: (i, 0)),
        out_shape=jax.ShapeDtypeStruct((e, 128), jnp.float32),
    )(ga, gc, es, ns8, nd8, ev3, wes, wvn1, w2s, b2, wvn2, w3s, b3,
      wvn3, wh1, wv1, wh2, wv2, wh3)


def kernel(x, normals, edge_s, edge_v, edge_index,
           wh1, ws1_w, ws1_b, wv1, wh2, ws2_w, ws2_b, wv2,
           wh3, ws3_w, ws3_b, wv3):
    n, din = x.shape
    e = edge_s.shape[0]
    wa = ws1_w[0:128]
    wes = ws1_w[128:144]
    wc = ws1_w[144:272]
    wvn1 = ws1_w[272:275]
    w2s, wvn2 = ws2_w[0:128], ws2_w[128:131]
    w3s, wvn3 = ws3_w[0:128], ws3_w[128:131]
    b1 = ws1_b.reshape(1, 128)
    b2 = ws2_b.reshape(1, 128)
    b3 = ws3_b.reshape(1, 128)

    ta, tc = _prep_call(x, wa, wc, b1)

    src = edge_index[0]
    dst = edge_index[1]
    np_ = ((n + 8 * _NS - 1) // (8 * _NS)) * (8 * _NS)

    nrm8 = jnp.pad(normals, ((0, 0), (0, 5)))
    zc = jnp.zeros((np_ // _NS, 8), jnp.float32)
    ones_in = jnp.ones((_CG, 8), jnp.float32)
    z128 = jnp.zeros((np_ // _NS, 128), jnp.float32)
    ev3 = edge_v.reshape(e, 3)

    h = e // 2
    parts, cnts = [], []
    msgs = []
    for lo in (0, h):
        s_h, d_h = src[lo:lo + h], dst[lo:lo + h]
        ga, gc = _gather_call(ta, tc, s_h, d_h)
        ns8, nd8, counts = _aux_call(nrm8, s_h, d_h, zc, ones_in, np_)
        msg = _dense_call(ga, gc, edge_s, ns8, nd8,
                          ev3, wes, wvn1, w2s, b2, wvn2, w3s, b3,
                          wvn3, wh1, wv1, wh2, wv2, wh3, lo // _BE)
        parts.append(_scatter_call(msg, d_h, z128, np_))
        cnts.append(counts)
    return _combine_call(parts[0], parts[1], cnts[0], cnts[1], n)
